# baseline (speedup 1.0000x reference)
import functools

import jax
import jax.numpy as jnp
from jax.experimental import pallas as pl
from jax.experimental.pallas import tpu as pltpu

_CHUNK = 200
_NBUF = 6


def _l_copy(L_ref, L_buf, l_sems, idx, slot, chunk):
    return pltpu.make_async_copy(
        L_ref.at[pl.ds(idx * chunk, chunk), :], L_buf.at[slot], l_sems.at[slot]
    )


def _out_copy(out_ref, out_buf, o_sems, idx, slot, chunk):
    return pltpu.make_async_copy(
        out_buf.at[slot], out_ref.at[pl.ds(idx * chunk, chunk), :], o_sems.at[slot]
    )


def _fused_body(L_ref, F_ref, Wc_ref, bc_ref, out_ref,
                L_buf, l_sems, out_buf, o_sems, *, chunk, nbuf, nchunks):
    for s in range(nbuf):
        _l_copy(L_ref, L_buf, l_sems, s, s, chunk).start()

    def step(j, carry):
        slot = jax.lax.rem(j, nbuf)
        _l_copy(L_ref, L_buf, l_sems, j, slot, chunk).wait()
        x = jnp.dot(L_buf[slot], F_ref[...], preferred_element_type=jnp.float32)
        f_row = F_ref[pl.ds(j * chunk, chunk), :]
        lhs = jnp.concatenate([f_row + x, x * f_row], axis=1)
        res = (
            jnp.dot(lhs, Wc_ref[...], preferred_element_type=jnp.float32)
            + bc_ref[...]
        )

        @pl.when(j >= nbuf)
        def _wait_out():
            _out_copy(out_ref, out_buf, o_sems, j - nbuf, slot, chunk).wait()

        out_buf[slot] = res
        _out_copy(out_ref, out_buf, o_sems, j, slot, chunk).start()

        @pl.when(j + nbuf < nchunks)
        def _next_l():
            _l_copy(L_ref, L_buf, l_sems, j + nbuf, slot, chunk).start()

        return carry

    jax.lax.fori_loop(0, nchunks, step, 0)

    for s in range(nbuf):
        idx = nchunks - nbuf + s
        _out_copy(out_ref, out_buf, o_sems, idx, idx % nbuf, chunk).wait()


def kernel(lap_matrix, eye_matrix, features, W1, b1, W2, b2):
    n, d = features.shape
    chunk, nbuf = _CHUNK, _NBUF
    nchunks = n // chunk

    Wc = jnp.concatenate([W1.T, W2.T], axis=0)
    bc = (b1 + b2).reshape(1, d)

    body = functools.partial(_fused_body, chunk=chunk, nbuf=nbuf, nchunks=nchunks)
    return pl.pallas_call(
        body,
        in_specs=[
            pl.BlockSpec(memory_space=pltpu.MemorySpace.HBM),
            pl.BlockSpec((n, d), lambda: (0, 0)),
            pl.BlockSpec((2 * d, d), lambda: (0, 0)),
            pl.BlockSpec((1, d), lambda: (0, 0)),
        ],
        out_specs=pl.BlockSpec(memory_space=pltpu.MemorySpace.HBM),
        out_shape=jax.ShapeDtypeStruct((n, d), jnp.float32),
        scratch_shapes=[
            pltpu.VMEM((nbuf, chunk, n), jnp.float32),
            pltpu.SemaphoreType.DMA((nbuf,)),
            pltpu.VMEM((nbuf, chunk, d), jnp.float32),
            pltpu.SemaphoreType.DMA((nbuf,)),
        ],
    )(lap_matrix, features, Wc, bc)


# speedup vs baseline: 1.0339x; 1.0339x over previous
import functools

import jax
import jax.numpy as jnp
from jax.experimental import pallas as pl
from jax.experimental.pallas import tpu as pltpu

_CHUNK = 400
_NBUF = 2


def _l_copy(L_ref, L_buf, l_sems, idx, slot, chunk):
    return pltpu.make_async_copy(
        L_ref.at[pl.ds(idx * chunk, chunk), :], L_buf.at[slot], l_sems.at[slot]
    )


def _out_copy(out_ref, out_buf, o_sems, idx, slot, chunk):
    return pltpu.make_async_copy(
        out_buf.at[slot], out_ref.at[pl.ds(idx * chunk, chunk), :], o_sems.at[slot]
    )


def _fused_body(L_ref, F_ref, Wc_ref, bc_ref, out_ref,
                L_buf, l_sems, out_buf, o_sems, *, chunk, nbuf, nchunks):
    for s in range(nbuf):
        _l_copy(L_ref, L_buf, l_sems, s, s, chunk).start()

    def step(j, carry):
        slot = jax.lax.rem(j, nbuf)
        _l_copy(L_ref, L_buf, l_sems, j, slot, chunk).wait()
        x = jnp.dot(L_buf[slot], F_ref[...], preferred_element_type=jnp.float32)
        f_row = F_ref[pl.ds(j * chunk, chunk), :]
        lhs = jnp.concatenate([f_row + x, x * f_row], axis=1)
        res = (
            jnp.dot(lhs, Wc_ref[...], preferred_element_type=jnp.float32)
            + bc_ref[...]
        )

        @pl.when(j >= nbuf)
        def _wait_out():
            _out_copy(out_ref, out_buf, o_sems, j - nbuf, slot, chunk).wait()

        out_buf[slot] = res
        _out_copy(out_ref, out_buf, o_sems, j, slot, chunk).start()

        @pl.when(j + nbuf < nchunks)
        def _next_l():
            _l_copy(L_ref, L_buf, l_sems, j + nbuf, slot, chunk).start()

        return carry

    jax.lax.fori_loop(0, nchunks, step, 0)

    for s in range(nbuf):
        idx = nchunks - nbuf + s
        _out_copy(out_ref, out_buf, o_sems, idx, idx % nbuf, chunk).wait()


def kernel(lap_matrix, eye_matrix, features, W1, b1, W2, b2):
    n, d = features.shape
    chunk, nbuf = _CHUNK, _NBUF
    nchunks = n // chunk

    Wc = jnp.concatenate([W1.T, W2.T], axis=0)
    bc = (b1 + b2).reshape(1, d)

    body = functools.partial(_fused_body, chunk=chunk, nbuf=nbuf, nchunks=nchunks)
    return pl.pallas_call(
        body,
        in_specs=[
            pl.BlockSpec(memory_space=pltpu.MemorySpace.HBM),
            pl.BlockSpec((n, d), lambda: (0, 0)),
            pl.BlockSpec((2 * d, d), lambda: (0, 0)),
            pl.BlockSpec((1, d), lambda: (0, 0)),
        ],
        out_specs=pl.BlockSpec(memory_space=pltpu.MemorySpace.HBM),
        out_shape=jax.ShapeDtypeStruct((n, d), jnp.float32),
        scratch_shapes=[
            pltpu.VMEM((nbuf, chunk, n), jnp.float32),
            pltpu.SemaphoreType.DMA((nbuf,)),
            pltpu.VMEM((nbuf, chunk, d), jnp.float32),
            pltpu.SemaphoreType.DMA((nbuf,)),
        ],
    )(lap_matrix, features, Wc, bc)
